# baseline (speedup 1.0000x reference)
import functools

import jax
import jax.numpy as jnp
from jax import lax
from jax.experimental import pallas as pl
from jax.experimental.pallas import tpu as pltpu
from jax.experimental.pallas import tpu_sc as plsc

NC, NS, LANES = 2, 16, 16
NW = NC * NS

B, L, D = 4, 8192, 768
P = 8192
N = B * L
TOK_PER_W = N // NW
CHUNK = 32
NCHUNK = TOK_PER_W // CHUNK
NJ = D // LANES
NACC = 4
GRP = 16
HALF = 8
LN_EPS = 1e-12


def _allsum16(v):
    iota = lax.iota(jnp.int32, LANES)
    for shift in (8, 4, 2, 1):
        idx = (iota + shift) & (LANES - 1)
        v = v + v.at[idx].get(mode="promise_in_bounds")
    return v


def _rsqrt16(v):
    i = lax.bitcast_convert_type(v, jnp.int32)
    i = jnp.int32(0x5F3759DF) - lax.shift_right_logical(i, 1)
    y = lax.bitcast_convert_type(i, jnp.float32)
    for _ in range(3):
        y = y * (1.5 - 0.5 * v * y * y)
    return y


def _sc_embed(ids, pos, tt, word_table, ptt_table, ln_w, ln_b):
    mesh = plsc.VectorSubcoreMesh(core_axis_name="c", subcore_axis_name="s")

    @functools.partial(
        pl.kernel,
        mesh=mesh,
        out_type=jax.ShapeDtypeStruct((N, D), jnp.float32),
        scratch_types=[
            pltpu.VMEM((TOK_PER_W,), jnp.int32),
            pltpu.VMEM((TOK_PER_W,), jnp.int32),
            pltpu.VMEM((TOK_PER_W,), jnp.int32),
            pltpu.VMEM((CHUNK, D), jnp.float32),
            pltpu.VMEM((CHUNK, D), jnp.float32),
            pltpu.VMEM((CHUNK, D), jnp.float32),
            pltpu.VMEM((CHUNK, D), jnp.float32),
            pltpu.VMEM((D,), jnp.float32),
            pltpu.VMEM((D,), jnp.float32),
            pltpu.SemaphoreType.DMA,
            pltpu.SemaphoreType.DMA,
            pltpu.SemaphoreType.DMA,
            pltpu.SemaphoreType.DMA,
            pltpu.SemaphoreType.DMA,
            pltpu.SemaphoreType.DMA,
            pltpu.SemaphoreType.DMA,
            pltpu.SemaphoreType.DMA,
            pltpu.SemaphoreType.DMA,
            pltpu.SemaphoreType.DMA,
            pltpu.SemaphoreType.DMA,
            pltpu.SemaphoreType.DMA,
        ],
    )
    def k(ids_hbm, pos_hbm, tt_hbm, word_hbm, ptt_hbm, lnw_hbm, lnb_hbm,
          out_hbm, idw_all, idp_all, idt_all,
          rw0, rw1, rp0, rp1, lnw_v, lnb_v,
          sw0, sw1, sw2, sw3, sp0, sp1, sp2, sp3, so0, so1, so2, so3):
        wid = lax.axis_index("s") * NC + lax.axis_index("c")
        wbase = wid * TOK_PER_W

        rw = (rw0, rw1)
        rp = (rp0, rp1)
        sw = (sw0, sw1, sw2, sw3)
        sp = (sp0, sp1, sp2, sp3)
        so = (so0, so1, so2, so3)

        pltpu.sync_copy(lnw_hbm, lnw_v)
        pltpu.sync_copy(lnb_hbm, lnb_v)
        pltpu.sync_copy(ids_hbm.at[pl.ds(wbase, TOK_PER_W)], idw_all)
        pltpu.sync_copy(pos_hbm.at[pl.ds(wbase, TOK_PER_W)], idp_all)
        pltpu.sync_copy(tt_hbm.at[pl.ds(wbase, TOK_PER_W)], idt_all)
        for q in range(TOK_PER_W // LANES):
            qs = pl.ds(q * LANES, LANES)
            idp_all[qs] = idp_all[qs] + idt_all[qs] * P

        lane_iota = lax.iota(jnp.int32, LANES)

        def issue(g, s):
            b, hoff = s // 2, (s % 2) * GRP
            for q in range(2):
                off = pl.ds(g * GRP + q * (GRP // 2), GRP // 2)
                half = pl.ds(hoff + q * (GRP // 2), GRP // 2)
                pltpu.async_copy(word_hbm.at[idw_all.at[off]],
                                 rw[b].at[half], sw[s])
                pltpu.async_copy(ptt_hbm.at[idp_all.at[off]],
                                 rp[b].at[half], sp[s])

        def wait(g, s):
            b, hoff = s // 2, (s % 2) * GRP
            off = pl.ds(g * GRP, GRP)
            half = pl.ds(hoff, GRP)
            pltpu.make_async_copy(word_hbm.at[idw_all.at[off]],
                                  rw[b].at[half], sw[s]).wait()
            pltpu.make_async_copy(ptt_hbm.at[idp_all.at[off]],
                                  rp[b].at[half], sp[s]).wait()

        def compute16(rw_v, rp_v, toff):
            def tok_body(t, carry):
                mv_c, ev_c = carry
                acc = [jnp.zeros((LANES,), jnp.float32) for _ in range(NACC)]
                acc2 = [jnp.zeros((LANES,), jnp.float32) for _ in range(NACC)]
                for j in range(NJ):
                    sl = pl.ds(j * LANES, LANES)
                    x = rw_v[t, sl] + rp_v[t, sl]
                    rw_v[t, sl] = x
                    acc[j % NACC] = acc[j % NACC] + x
                    acc2[j % NACC] = acc2[j % NACC] + x * x
                a = (acc[0] + acc[1]) + (acc[2] + acc[3])
                a2 = (acc2[0] + acc2[1]) + (acc2[2] + acc2[3])
                asum = _allsum16(a)
                a2sum = _allsum16(a2)
                msk = lane_iota == (t - toff)
                return (jnp.where(msk, asum, mv_c),
                        jnp.where(msk, a2sum, ev_c))

            zz = jnp.zeros((LANES,), jnp.float32)
            mv, ev = lax.fori_loop(toff, toff + GRP, tok_body, (zz, zz))

            mean = mv * (1.0 / D)
            ex2 = ev * (1.0 / D)
            var = ex2 - mean * mean
            inv = _rsqrt16(var + LN_EPS)
            a16 = inv
            c16 = mean * inv

            for h in range(GRP // HALF):
                toks = range(toff + h * HALF, toff + (h + 1) * HALF)
                As = []
                Cs = []
                for t in toks:
                    tsplat = lane_iota * 0 + (t - toff)
                    As.append(a16.at[tsplat].get(mode="promise_in_bounds"))
                    Cs.append(c16.at[tsplat].get(mode="promise_in_bounds"))

                def norm_body(j, nc):
                    sl = pl.ds(j * LANES, LANES)
                    w = lnw_v[sl]
                    bb = lnb_v[sl]
                    for i, t in enumerate(toks):
                        x = rw_v[t, sl]
                        rw_v[t, sl] = (x * As[i] - Cs[i]) * w + bb
                    return nc

                lax.fori_loop(0, NJ, norm_body, 0)

        NGR = TOK_PER_W // GRP

        def compute(g, s):
            b, hoff = s // 2, (s % 2) * GRP
            compute16(rw[b], rp[b], hoff)
            pltpu.async_copy(rw[b].at[pl.ds(hoff, GRP)],
                             out_hbm.at[pl.ds(wbase + g * GRP, GRP)],
                             so[s])

        def wait_out(s):
            b, hoff = s // 2, (s % 2) * GRP
            pltpu.make_async_copy(rw[b].at[pl.ds(hoff, GRP)],
                                  out_hbm.at[pl.ds(wbase, GRP)],
                                  so[s]).wait()

        issue(0, 0)
        issue(1, 1)
        issue(2, 2)

        def quad_body(i, carry):
            g0 = 4 * i
            for s in range(4):
                g = g0 + s
                wait(g, s)
                compute(g, s)
                nxt = (s + 3) % 4

                @pl.when(jnp.logical_and(g + 3 < NGR, g >= 1))
                def _():
                    wait_out(nxt)

                @pl.when(g + 3 < NGR)
                def _():
                    issue(g + 3, nxt)

            return carry

        lax.fori_loop(0, NGR // 4, quad_body, 0)
        for s in range(4):
            wait_out(s)

    return k(ids, pos, tt, word_table, ptt_table, ln_w, ln_b)


def kernel(input_ids, token_type_ids, position_ids, word_table, pos_table,
           type_table, ln_w, ln_b):
    ids = jnp.asarray(input_ids, jnp.int32).reshape(N)
    pos = jnp.asarray(position_ids, jnp.int32).reshape(N)
    tt = jnp.asarray(token_type_ids, jnp.int32).reshape(N)
    ptt = (pos_table.astype(jnp.float32)[None, :, :]
           + type_table.astype(jnp.float32)[:, None, :]).reshape(2 * P, D)
    out = _sc_embed(ids, pos, tt,
                    word_table.astype(jnp.float32), ptt,
                    ln_w.astype(jnp.float32), ln_b.astype(jnp.float32))
    return out.reshape(B, L, D)


# speedup vs baseline: 1.0060x; 1.0060x over previous
import functools

import jax
import jax.numpy as jnp
from jax import lax
from jax.experimental import pallas as pl
from jax.experimental.pallas import tpu as pltpu
from jax.experimental.pallas import tpu_sc as plsc

NC, NS, LANES = 2, 16, 16
NW = NC * NS

B, L, D = 4, 8192, 768
P = 8192
N = B * L
TOK_PER_W = N // NW
CHUNK = 32
NCHUNK = TOK_PER_W // CHUNK
NJ = D // LANES
NACC = 4
GRP = 16
HALF = 8
LN_EPS = 1e-12


def _allsum16(v):
    iota = lax.iota(jnp.int32, LANES)
    for shift in (8, 4, 2, 1):
        idx = (iota + shift) & (LANES - 1)
        v = v + v.at[idx].get(mode="promise_in_bounds")
    return v


def _rsqrt16(v):
    i = lax.bitcast_convert_type(v, jnp.int32)
    i = jnp.int32(0x5F3759DF) - lax.shift_right_logical(i, 1)
    y = lax.bitcast_convert_type(i, jnp.float32)
    for _ in range(3):
        y = y * (1.5 - 0.5 * v * y * y)
    return y


def _sc_embed(ids, pos, tt, word_table, ptt_table, ln_w, ln_b):
    mesh = plsc.VectorSubcoreMesh(core_axis_name="c", subcore_axis_name="s")

    @functools.partial(
        pl.kernel,
        mesh=mesh,
        out_type=jax.ShapeDtypeStruct((N, D), jnp.float32),
        scratch_types=[
            pltpu.VMEM((TOK_PER_W,), jnp.int32),
            pltpu.VMEM((TOK_PER_W,), jnp.int32),
            pltpu.VMEM((TOK_PER_W,), jnp.int32),
            pltpu.VMEM((CHUNK, D), jnp.float32),
            pltpu.VMEM((CHUNK, D), jnp.float32),
            pltpu.VMEM((CHUNK, D), jnp.float32),
            pltpu.VMEM((CHUNK, D), jnp.float32),
            pltpu.VMEM((D,), jnp.float32),
            pltpu.VMEM((D,), jnp.float32),
            pltpu.SemaphoreType.DMA,
            pltpu.SemaphoreType.DMA,
            pltpu.SemaphoreType.DMA,
            pltpu.SemaphoreType.DMA,
            pltpu.SemaphoreType.DMA,
            pltpu.SemaphoreType.DMA,
            pltpu.SemaphoreType.DMA,
            pltpu.SemaphoreType.DMA,
            pltpu.SemaphoreType.DMA,
            pltpu.SemaphoreType.DMA,
            pltpu.SemaphoreType.DMA,
            pltpu.SemaphoreType.DMA,
        ],
    )
    def k(ids_hbm, pos_hbm, tt_hbm, word_hbm, ptt_hbm, lnw_hbm, lnb_hbm,
          out_hbm, idw_all, idp_all, idt_all,
          rw0, rw1, rp0, rp1, lnw_v, lnb_v,
          sw0, sw1, sw2, sw3, sp0, sp1, sp2, sp3, so0, so1, so2, so3):
        wid = lax.axis_index("s") * NC + lax.axis_index("c")
        wbase = wid * TOK_PER_W

        rw = (rw0, rw1)
        rp = (rp0, rp1)
        sw = (sw0, sw1, sw2, sw3)
        sp = (sp0, sp1, sp2, sp3)
        so = (so0, so1, so2, so3)

        pltpu.sync_copy(lnw_hbm, lnw_v)
        pltpu.sync_copy(lnb_hbm, lnb_v)
        pltpu.sync_copy(ids_hbm.at[pl.ds(wbase, TOK_PER_W)], idw_all)
        pltpu.sync_copy(pos_hbm.at[pl.ds(wbase, TOK_PER_W)], idp_all)
        pltpu.sync_copy(tt_hbm.at[pl.ds(wbase, TOK_PER_W)], idt_all)
        for q in range(TOK_PER_W // LANES):
            qs = pl.ds(q * LANES, LANES)
            idp_all[qs] = idp_all[qs] + idt_all[qs] * P

        lane_iota = lax.iota(jnp.int32, LANES)

        def issue(g, s):
            b, hoff = s // 2, (s % 2) * GRP
            off = pl.ds(g * GRP, GRP)
            half = pl.ds(hoff, GRP)
            pltpu.async_copy(word_hbm.at[idw_all.at[off]],
                             rw[b].at[half], sw[s])
            pltpu.async_copy(ptt_hbm.at[idp_all.at[off]],
                             rp[b].at[half], sp[s])

        def wait(g, s):
            b, hoff = s // 2, (s % 2) * GRP
            off = pl.ds(g * GRP, GRP)
            half = pl.ds(hoff, GRP)
            pltpu.make_async_copy(word_hbm.at[idw_all.at[off]],
                                  rw[b].at[half], sw[s]).wait()
            pltpu.make_async_copy(ptt_hbm.at[idp_all.at[off]],
                                  rp[b].at[half], sp[s]).wait()

        def compute16(rw_v, rp_v, toff):
            def tok_body(t, carry):
                mv_c, ev_c = carry
                acc = [jnp.zeros((LANES,), jnp.float32) for _ in range(NACC)]
                acc2 = [jnp.zeros((LANES,), jnp.float32) for _ in range(NACC)]
                for j in range(NJ):
                    sl = pl.ds(j * LANES, LANES)
                    x = rw_v[t, sl] + rp_v[t, sl]
                    rw_v[t, sl] = x
                    acc[j % NACC] = acc[j % NACC] + x
                    acc2[j % NACC] = acc2[j % NACC] + x * x
                a = (acc[0] + acc[1]) + (acc[2] + acc[3])
                a2 = (acc2[0] + acc2[1]) + (acc2[2] + acc2[3])
                asum = _allsum16(a)
                a2sum = _allsum16(a2)
                msk = lane_iota == (t - toff)
                return (jnp.where(msk, asum, mv_c),
                        jnp.where(msk, a2sum, ev_c))

            zz = jnp.zeros((LANES,), jnp.float32)
            mv, ev = lax.fori_loop(toff, toff + GRP, tok_body, (zz, zz))

            mean = mv * (1.0 / D)
            ex2 = ev * (1.0 / D)
            var = ex2 - mean * mean
            inv = _rsqrt16(var + LN_EPS)
            a16 = inv
            c16 = mean * inv

            for h in range(GRP // HALF):
                toks = range(toff + h * HALF, toff + (h + 1) * HALF)
                As = []
                Cs = []
                for t in toks:
                    tsplat = lane_iota * 0 + (t - toff)
                    As.append(a16.at[tsplat].get(mode="promise_in_bounds"))
                    Cs.append(c16.at[tsplat].get(mode="promise_in_bounds"))

                def norm_body(j, nc):
                    sl = pl.ds(j * LANES, LANES)
                    w = lnw_v[sl]
                    bb = lnb_v[sl]
                    for i, t in enumerate(toks):
                        x = rw_v[t, sl]
                        rw_v[t, sl] = (x * As[i] - Cs[i]) * w + bb
                    return nc

                lax.fori_loop(0, NJ, norm_body, 0)

        NGR = TOK_PER_W // GRP

        def compute(g, s):
            b, hoff = s // 2, (s % 2) * GRP
            compute16(rw[b], rp[b], hoff)
            pltpu.async_copy(rw[b].at[pl.ds(hoff, GRP)],
                             out_hbm.at[pl.ds(wbase + g * GRP, GRP)],
                             so[s])

        def wait_out(s):
            b, hoff = s // 2, (s % 2) * GRP
            pltpu.make_async_copy(rw[b].at[pl.ds(hoff, GRP)],
                                  out_hbm.at[pl.ds(wbase, GRP)],
                                  so[s]).wait()

        issue(0, 0)
        issue(1, 1)
        issue(2, 2)

        def quad_body(i, carry):
            g0 = 4 * i
            for s in range(4):
                g = g0 + s
                wait(g, s)
                compute(g, s)
                nxt = (s + 3) % 4

                @pl.when(jnp.logical_and(g + 3 < NGR, g >= 1))
                def _():
                    wait_out(nxt)

                @pl.when(g + 3 < NGR)
                def _():
                    issue(g + 3, nxt)

            return carry

        lax.fori_loop(0, NGR // 4, quad_body, 0)
        for s in range(4):
            wait_out(s)

    return k(ids, pos, tt, word_table, ptt_table, ln_w, ln_b)


def kernel(input_ids, token_type_ids, position_ids, word_table, pos_table,
           type_table, ln_w, ln_b):
    ids = jnp.asarray(input_ids, jnp.int32).reshape(N)
    pos = jnp.asarray(position_ids, jnp.int32).reshape(N)
    tt = jnp.asarray(token_type_ids, jnp.int32).reshape(N)
    ptt = (pos_table.astype(jnp.float32)[None, :, :]
           + type_table.astype(jnp.float32)[:, None, :]).reshape(2 * P, D)
    out = _sc_embed(ids, pos, tt,
                    word_table.astype(jnp.float32), ptt,
                    ln_w.astype(jnp.float32), ln_b.astype(jnp.float32))
    return out.reshape(B, L, D)
